# baseline (speedup 1.0000x reference)
import functools

import jax
import jax.numpy as jnp
from jax import lax
from jax.experimental import pallas as pl
from jax.experimental.pallas import tpu as pltpu
from jax.experimental.pallas import tpu_sc as plsc

_INFO = plsc.get_sparse_core_info()
_NC = _INFO.num_cores
_NS = _INFO.num_subcores
_NW = _NC * _NS
_L = _INFO.num_lanes

_B = 16384
_D = 64
_BPW = _B // _NW
_CHUNK = 128
_NCHUNK = _BPW // _CHUNK
_DC = _D // _L
_NGRP = _BPW // _L


def _body(uidx_h, iidx_h, utab_h, itab_h, w_h, b_h, out_h,
          uidx_v, iidx_v, urows_v, irows_v, w_v, b_v, out_v, sem):
    wid = lax.axis_index("s") * _NC + lax.axis_index("c")

    pltpu.sync_copy(uidx_h.at[wid], uidx_v)
    pltpu.sync_copy(iidx_h.at[wid], iidx_v)

    copies = []
    for j in range(_NCHUNK):
        dst = pl.ds(j * _CHUNK, _CHUNK)
        copies.append(pltpu.async_copy(utab_h.at[uidx_v.at[j]],
                                       urows_v.at[dst], sem))
        copies.append(pltpu.async_copy(itab_h.at[iidx_v.at[j]],
                                       irows_v.at[dst], sem))

    pltpu.sync_copy(w_h, w_v)
    pltpu.sync_copy(b_h, b_v)
    for c in copies:
        c.wait()

    wc = [w_v[c] for c in range(_DC)]
    bvec = b_v[:]
    iota = lax.iota(jnp.int32, _L)
    zero = jnp.zeros((_L,), jnp.float32)

    def group(g, carry):
        acc = bvec
        for r in range(_L):
            row = g * _L + r
            s = None
            for c in range(_DC):
                u = urows_v[row, pl.ds(c * _L, _L)]
                v = irows_v[row, pl.ds(c * _L, _L)]
                t = u * v * wc[c]
                s = t if s is None else s + t
            tot = jnp.sum(s)
            acc = acc + jnp.where(iota == r, tot, zero)
        rating = 1.0 / (1.0 + jnp.exp(-acc))
        out_v[pl.ds(g * _L, _L)] = rating
        return carry

    lax.fori_loop(0, _NGRP, group, 0)
    pltpu.sync_copy(out_v, out_h.at[wid])


@jax.jit
def _gmf(uidx, iidx, utab, itab, w, b):
    mesh = plsc.VectorSubcoreMesh(core_axis_name="c", subcore_axis_name="s")
    return pl.kernel(
        _body,
        out_type=jax.ShapeDtypeStruct((_NW, _BPW), jnp.float32),
        mesh=mesh,
        compiler_params=pltpu.CompilerParams(
            needs_layout_passes=False, use_tc_tiling_on_sc=False),
        scratch_types=[
            pltpu.VMEM((_NCHUNK, _CHUNK), jnp.int32),
            pltpu.VMEM((_NCHUNK, _CHUNK), jnp.int32),
            pltpu.VMEM((_BPW, _D), jnp.float32),
            pltpu.VMEM((_BPW, _D), jnp.float32),
            pltpu.VMEM((_DC, _L), jnp.float32),
            pltpu.VMEM((_L,), jnp.float32),
            pltpu.VMEM((_BPW,), jnp.float32),
            pltpu.SemaphoreType.DMA,
        ],
    )(uidx, iidx, utab, itab, w, b)


def kernel(user_indices, item_indices, user_table, item_table, affine_w, affine_b):
    uidx = user_indices.astype(jnp.int32).reshape(_NW, _NCHUNK, _CHUNK)
    iidx = item_indices.astype(jnp.int32).reshape(_NW, _NCHUNK, _CHUNK)
    w = affine_w.reshape(_DC, _L)
    b = jnp.broadcast_to(affine_b.reshape(1), (_L,))
    out = _gmf(uidx, iidx, user_table, item_table, w, b)
    return out.reshape(_B, 1)
